# baseline (speedup 1.0000x reference)
import functools

import jax
import jax.numpy as jnp
from jax import lax
from jax.experimental import pallas as pl
from jax.experimental.pallas import tpu as pltpu, tpu_sc as plsc

NUM_CORES = 2
NUM_SUBCORES = 16
LANES = 16
NW = NUM_CORES * NUM_SUBCORES

BATCH = 16384
EMBED = 64
PACK = 2 * EMBED
B_PER_W = BATCH // NW
CHUNK = 128
NCHUNK = B_PER_W // CHUNK


def _make_kernel():
    mesh = plsc.VectorSubcoreMesh(core_axis_name="c", subcore_axis_name="s")

    @functools.partial(
        pl.kernel,
        mesh=mesh,
        out_type=jax.ShapeDtypeStruct((NW, B_PER_W), jnp.float32),
        scratch_types=[
            pltpu.VMEM((B_PER_W + LANES,), jnp.int32),
            pltpu.VMEM((B_PER_W + LANES,), jnp.int32),
            pltpu.VMEM((NCHUNK, CHUNK), jnp.int32),
            pltpu.VMEM((NCHUNK, CHUNK), jnp.int32),
            pltpu.VMEM((2, CHUNK, PACK), jnp.float32),
            pltpu.VMEM((2, CHUNK, PACK), jnp.float32),
            pltpu.VMEM((B_PER_W,), jnp.float32),
            pltpu.SemaphoreType.DMA,
            pltpu.SemaphoreType.DMA,
        ],
    )
    def dot_kernel(uids_hbm, iids_hbm, utab_hbm, itab_hbm, out_hbm,
                   uid_v, iid_v, upk_v, ipk_v,
                   urows_v, irows_v, out_v, sem0, sem1):
        wid = lax.axis_index("s") * NUM_CORES + lax.axis_index("c")

        pltpu.sync_copy(uids_hbm.at[wid], uid_v.at[pl.ds(0, B_PER_W)])
        pltpu.sync_copy(iids_hbm.at[wid], iid_v.at[pl.ds(0, B_PER_W)])

        def pk_body(t, carry):
            ids_u = uid_v[pl.ds(t * LANES, LANES)]
            ids_i = iid_v[pl.ds(t * LANES, LANES)]
            c = t // (CHUNK // LANES)
            o = (t % (CHUNK // LANES)) * LANES
            upk_v[c, pl.ds(o, LANES)] = lax.shift_right_logical(ids_u, 1)
            ipk_v[c, pl.ds(o, LANES)] = lax.shift_right_logical(ids_i, 1)
            return carry

        lax.fori_loop(0, B_PER_W // LANES, pk_body, 0)

        sems = (sem0, sem1)

        def fire(c):
            sem = sems[c % 2]
            buf = c % 2
            return [
                pltpu.async_copy(utab_hbm.at[upk_v.at[c]], urows_v.at[buf], sem),
                pltpu.async_copy(itab_hbm.at[ipk_v.at[c]], irows_v.at[buf], sem),
            ]

        lane_ids = lax.iota(jnp.int32, LANES)
        perms = [(lane_ids + sh) % LANES for sh in (8, 4, 2, 1)]
        dnums = lax.GatherDimensionNumbers(
            offset_dims=(), collapsed_slice_dims=(0,), start_index_map=(0,))

        def lane_sum(x):
            for perm in perms:
                rot = lax.gather(
                    x, perm[:, None], dnums, (1,),
                    mode=lax.GatherScatterMode.PROMISE_IN_BOUNDS)
                x = x + rot
            return x

        def compute(c):
            buf = c % 2

            def group_body(g, carry):
                def row_body(j, acc):
                    b = g * LANES + j
                    uid_b = uid_v[pl.ds(c * CHUNK + b, LANES)][0]
                    iid_b = iid_v[pl.ds(c * CHUNK + b, LANES)][0]
                    uoff = jnp.bitwise_and(uid_b, 1) * EMBED
                    ioff = jnp.bitwise_and(iid_b, 1) * EMBED
                    s = None
                    for q in range(EMBED // LANES):
                        u = urows_v[buf, b, pl.ds(uoff + q * LANES, LANES)]
                        v = irows_v[buf, b, pl.ds(ioff + q * LANES, LANES)]
                        p = u * v
                        s = p if s is None else s + p
                    dot = lane_sum(s)
                    return jnp.where(lane_ids == j, dot, acc)

                accv = lax.fori_loop(0, LANES, row_body,
                                     jnp.zeros((LANES,), jnp.float32))
                out_v[pl.ds(c * CHUNK + g * LANES, LANES)] = accv
                return carry

            lax.fori_loop(0, CHUNK // LANES, group_body, 0)

        inflight = fire(0)
        for c in range(NCHUNK):
            if c + 1 < NCHUNK:
                nxt = fire(c + 1)
            for cp in inflight:
                cp.wait()
            compute(c)
            if c + 1 < NCHUNK:
                inflight = nxt

        pltpu.sync_copy(out_v, out_hbm.at[wid])

    return dot_kernel


@jax.jit
def kernel(user_ids, item_ids, user_table, item_table):
    uids = user_ids.astype(jnp.int32).reshape(NW, B_PER_W)
    iids = item_ids.astype(jnp.int32).reshape(NW, B_PER_W)
    ut2 = user_table.reshape(user_table.shape[0] // 2, PACK)
    it2 = item_table.reshape(item_table.shape[0] // 2, PACK)
    fn = _make_kernel()
    out = fn(uids, iids, ut2, it2)
    return out.reshape(BATCH)
